# baseline (speedup 1.0000x reference)
import jax
import jax.numpy as jnp
from jax.experimental import pallas as pl
from jax.experimental.pallas import tpu as pltpu

_NCHUNK = 4


def _vq_body(x_ref, w_ref, q_ref, idx_ref):
    x = x_ref[0]
    w = w_ref[...]
    K = w.shape[0]
    KC = K // _NCHUNK

    xn = jnp.sum(x * x, axis=0, keepdims=True)
    fiota = jax.lax.broadcasted_iota(jnp.int32, (KC, 1), 0).astype(jnp.float32)

    m = None
    for c in range(_NCHUNK):
        wc = w[c * KC:(c + 1) * KC]
        s2 = jax.lax.dot_general(
            -2.0 * wc, x, (((1,), (0,)), ((), ())),
            preferred_element_type=jnp.float32)
        wn = jnp.sum(wc * wc, axis=1, keepdims=True)
        dist = (wn + xn) + s2
        mc = jnp.min(dist, axis=0, keepdims=True)
        fc = jnp.min(jnp.where(dist == mc, fiota + float(c * KC), float(K)),
                     axis=0, keepdims=True)
        if m is None:
            m, fidx = mc, fc
        else:
            take = mc < m
            m = jnp.where(take, mc, m)
            fidx = jnp.where(take, fc, fidx)

    q = None
    for c in range(_NCHUNK):
        oh = jnp.where(fiota + float(c * KC) == fidx, 1.0, 0.0)
        qc = jax.lax.dot_general(
            w[c * KC:(c + 1) * KC], oh, (((0,), (0,)), ((), ())),
            preferred_element_type=jnp.float32)
        q = qc if q is None else q + qc

    q_ref[0] = x + (q - x)
    idx_ref[0] = fidx.astype(jnp.int32)


_TT = 1024


def kernel(inputs, W):
    B, D, T = inputs.shape
    K = W.shape[0]
    nt = T // _TT
    q, idx = pl.pallas_call(
        _vq_body,
        grid=(B, nt),
        in_specs=[
            pl.BlockSpec((1, D, _TT), lambda b, j: (b, 0, j)),
            pl.BlockSpec((K, D), lambda b, j: (0, 0)),
        ],
        out_specs=[
            pl.BlockSpec((1, D, _TT), lambda b, j: (b, 0, j)),
            pl.BlockSpec((1, 1, _TT), lambda b, j: (b, 0, j)),
        ],
        out_shape=[
            jax.ShapeDtypeStruct((B, D, T), jnp.float32),
            jax.ShapeDtypeStruct((B, 1, T), jnp.int32),
        ],
        compiler_params=pltpu.CompilerParams(
            dimension_semantics=("parallel", "parallel")),
    )(inputs, W)
    return (q, idx.reshape(B * T, 1))


# speedup vs baseline: 1.0622x; 1.0622x over previous
import jax
import jax.numpy as jnp
from jax.experimental import pallas as pl
from jax.experimental.pallas import tpu as pltpu

_NCHUNK = 4


_BB = 2


def _vq_body(x_ref, w_ref, q_ref, idx_ref):
    for i in range(_BB):
        _vq_one(x_ref[i], w_ref[...], q_ref, idx_ref, i)


def _vq_one(x, w, q_ref, idx_ref, i):
    K = w.shape[0]
    KC = K // _NCHUNK

    xn = jnp.sum(x * x, axis=0, keepdims=True)
    fiota = jax.lax.broadcasted_iota(jnp.int32, (KC, 1), 0).astype(jnp.float32)

    m = None
    for c in range(_NCHUNK):
        wc = w[c * KC:(c + 1) * KC]
        s2 = jax.lax.dot_general(
            -2.0 * wc, x, (((1,), (0,)), ((), ())),
            preferred_element_type=jnp.float32)
        wn = jnp.sum(wc * wc, axis=1, keepdims=True)
        dist = (wn + xn) + s2
        mc = jnp.min(dist, axis=0, keepdims=True)
        fc = jnp.min(jnp.where(dist == mc, fiota + float(c * KC), float(K)),
                     axis=0, keepdims=True)
        if m is None:
            m, fidx = mc, fc
        else:
            take = mc < m
            m = jnp.where(take, mc, m)
            fidx = jnp.where(take, fc, fidx)

    q = None
    for c in range(_NCHUNK):
        oh = jnp.where(fiota + float(c * KC) == fidx, 1.0, 0.0)
        qc = jax.lax.dot_general(
            w[c * KC:(c + 1) * KC], oh, (((0,), (0,)), ((), ())),
            preferred_element_type=jnp.float32)
        q = qc if q is None else q + qc

    q_ref[i] = x + (q - x)
    idx_ref[i] = fidx.astype(jnp.int32)


_TT = 1024


def kernel(inputs, W):
    B, D, T = inputs.shape
    K = W.shape[0]
    nt = T // _TT
    q, idx = pl.pallas_call(
        _vq_body,
        grid=(B // _BB, nt),
        in_specs=[
            pl.BlockSpec((_BB, D, _TT), lambda b, j: (b, 0, j)),
            pl.BlockSpec((K, D), lambda b, j: (0, 0)),
        ],
        out_specs=[
            pl.BlockSpec((_BB, D, _TT), lambda b, j: (b, 0, j)),
            pl.BlockSpec((_BB, 1, _TT), lambda b, j: (b, 0, j)),
        ],
        out_shape=[
            jax.ShapeDtypeStruct((B, D, T), jnp.float32),
            jax.ShapeDtypeStruct((B, 1, T), jnp.int32),
        ],
        compiler_params=pltpu.CompilerParams(
            dimension_semantics=("parallel", "parallel")),
    )(inputs, W)
    return (q, idx.reshape(B * T, 1))


# speedup vs baseline: 1.0838x; 1.0204x over previous
import jax
import jax.numpy as jnp
from jax.experimental import pallas as pl
from jax.experimental.pallas import tpu as pltpu

_NCHUNK = 4


_BB = 4


def _vq_body(x_ref, w_ref, q_ref, idx_ref):
    for i in range(_BB):
        _vq_one(x_ref[i], w_ref[...], q_ref, idx_ref, i)


def _vq_one(x, w, q_ref, idx_ref, i):
    K = w.shape[0]
    KC = K // _NCHUNK

    xn = jnp.sum(x * x, axis=0, keepdims=True)
    fiota = jax.lax.broadcasted_iota(jnp.int32, (KC, 1), 0).astype(jnp.float32)

    m = None
    for c in range(_NCHUNK):
        wc = w[c * KC:(c + 1) * KC]
        s2 = jax.lax.dot_general(
            -2.0 * wc, x, (((1,), (0,)), ((), ())),
            preferred_element_type=jnp.float32)
        wn = jnp.sum(wc * wc, axis=1, keepdims=True)
        dist = (wn + xn) + s2
        mc = jnp.min(dist, axis=0, keepdims=True)
        fc = jnp.min(jnp.where(dist == mc, fiota + float(c * KC), float(K)),
                     axis=0, keepdims=True)
        if m is None:
            m, fidx = mc, fc
        else:
            take = mc < m
            m = jnp.where(take, mc, m)
            fidx = jnp.where(take, fc, fidx)

    q = None
    for c in range(_NCHUNK):
        oh = jnp.where(fiota + float(c * KC) == fidx, 1.0, 0.0)
        qc = jax.lax.dot_general(
            w[c * KC:(c + 1) * KC], oh, (((0,), (0,)), ((), ())),
            preferred_element_type=jnp.float32)
        q = qc if q is None else q + qc

    q_ref[i] = x + (q - x)
    idx_ref[i] = fidx.astype(jnp.int32)


_TT = 1024


def kernel(inputs, W):
    B, D, T = inputs.shape
    K = W.shape[0]
    nt = T // _TT
    q, idx = pl.pallas_call(
        _vq_body,
        grid=(B // _BB, nt),
        in_specs=[
            pl.BlockSpec((_BB, D, _TT), lambda b, j: (b, 0, j)),
            pl.BlockSpec((K, D), lambda b, j: (0, 0)),
        ],
        out_specs=[
            pl.BlockSpec((_BB, D, _TT), lambda b, j: (b, 0, j)),
            pl.BlockSpec((_BB, 1, _TT), lambda b, j: (b, 0, j)),
        ],
        out_shape=[
            jax.ShapeDtypeStruct((B, D, T), jnp.float32),
            jax.ShapeDtypeStruct((B, 1, T), jnp.int32),
        ],
        compiler_params=pltpu.CompilerParams(
            dimension_semantics=("parallel", "parallel")),
    )(inputs, W)
    return (q, idx.reshape(B * T, 1))
